# Initial kernel scaffold; baseline (speedup 1.0000x reference)
import jax
import jax.numpy as jnp
from jax.experimental import pallas as pl


def kernel(x, table):
    raise NotImplementedError("write your pallas kernel here")


# speedup vs baseline: 2.0398x; 2.0398x over previous
import functools

import jax
import jax.numpy as jnp
from jax import lax
from jax.experimental import pallas as pl
from jax.experimental.pallas import tpu as pltpu
from jax.experimental.pallas import tpu_sc as plsc

D_MODEL = 64
N_ROWS = 4096
N_COLS = 50
B_TOTAL = N_ROWS * N_COLS
NUM_WORKERS = 32
PER_WORKER = B_TOTAL // NUM_WORKERS
CHUNK = 128
N_CHUNKS = PER_WORKER // CHUNK
LANES = 16


def _tanh_lanes(z):
    e = jnp.exp(jnp.abs(z) * -2.0)
    r = (1.0 - e) / (1.0 + e)
    return jnp.sign(z) * r


def _emb_body(x_hbm, table_hbm, out_hbm, idx_v, rows_v, sem):
    wid = lax.axis_index("s") * 2 + lax.axis_index("c")
    base = wid * PER_WORKER

    def chunk_body(c, carry):
        row0 = base + c * CHUNK
        pltpu.sync_copy(x_hbm.at[pl.ds(row0, CHUNK)], idx_v)
        pltpu.async_copy(table_hbm.at[idx_v], rows_v, sem).wait()

        def row_body(i, carry2):
            for j in range(D_MODEL // LANES):
                z = rows_v[i, pl.ds(j * LANES, LANES)]
                rows_v[i, pl.ds(j * LANES, LANES)] = _tanh_lanes(z)
            return carry2

        lax.fori_loop(0, CHUNK, row_body, 0)
        pltpu.sync_copy(rows_v, out_hbm.at[pl.ds(row0, CHUNK)])
        return carry

    lax.fori_loop(0, N_CHUNKS, chunk_body, 0)


_emb = pl.kernel(
    _emb_body,
    out_type=jax.ShapeDtypeStruct((B_TOTAL, D_MODEL), jnp.float32),
    mesh=plsc.VectorSubcoreMesh(core_axis_name="c", subcore_axis_name="s"),
    scratch_types=[
        pltpu.VMEM((CHUNK,), jnp.int32),
        pltpu.VMEM((CHUNK, D_MODEL), jnp.float32),
        pltpu.SemaphoreType.DMA,
    ],
    compiler_params=pltpu.CompilerParams(use_tc_tiling_on_sc=False),
)


def kernel(x, table):
    xf = x.reshape(-1).astype(jnp.int32)
    out = _emb(xf, table)
    return out.reshape(N_ROWS, N_COLS, D_MODEL)


# speedup vs baseline: 2.4275x; 1.1901x over previous
import jax
import jax.numpy as jnp
from jax import lax
from jax.experimental import pallas as pl
from jax.experimental.pallas import tpu as pltpu
from jax.experimental.pallas import tpu_sc as plsc

D_MODEL = 64
N_ROWS = 4096
N_COLS = 50
B_TOTAL = N_ROWS * N_COLS
NUM_WORKERS = 32
PER_WORKER = B_TOTAL // NUM_WORKERS
CHUNK = 128
N_CHUNKS = PER_WORKER // CHUNK
LANES = 16
NBUF = 5
GROUPS = N_CHUNKS // NBUF


def _tanh_lanes(z):
    e = jnp.exp(jnp.abs(z) * -2.0)
    r = (1.0 - e) / (1.0 + e)
    return jnp.sign(z) * r


def _tanh_buf(buf):
    def row_body(i, carry):
        for j in range(D_MODEL // LANES):
            z = buf[i, pl.ds(j * LANES, LANES)]
            buf[i, pl.ds(j * LANES, LANES)] = _tanh_lanes(z)
        return carry

    lax.fori_loop(0, CHUNK, row_body, 0)


def _emb_body(x_hbm, table_hbm, out_hbm, idx_all, bufs, gsems, wsems):
    wid = lax.axis_index("s") * 2 + lax.axis_index("c")
    base = wid * PER_WORKER

    pltpu.sync_copy(x_hbm.at[pl.ds(wid * N_CHUNKS, N_CHUNKS)], idx_all)

    for b in range(NBUF):
        pltpu.async_copy(table_hbm.at[idx_all.at[b]], bufs[b], gsems[b])

    def group_body(g, carry):
        for b in range(NBUF):
            c = g * NBUF + b
            pltpu.make_async_copy(
                table_hbm.at[idx_all.at[c]], bufs[b], gsems[b]).wait()
            _tanh_buf(bufs[b])
            pltpu.async_copy(
                bufs[b], out_hbm.at[pl.ds(base + c * CHUNK, CHUNK)], wsems[b])

            @pl.when(g < GROUPS - 1)
            def _():
                pltpu.make_async_copy(
                    bufs[b], out_hbm.at[pl.ds(base + c * CHUNK, CHUNK)],
                    wsems[b]).wait()
                pltpu.async_copy(
                    table_hbm.at[idx_all.at[c + NBUF]], bufs[b], gsems[b])

        return carry

    lax.fori_loop(0, GROUPS, group_body, 0)

    for b in range(NBUF):
        c = (GROUPS - 1) * NBUF + b
        pltpu.make_async_copy(
            bufs[b], out_hbm.at[pl.ds(base + c * CHUNK, CHUNK)], wsems[b]).wait()


_emb = pl.kernel(
    _emb_body,
    out_type=jax.ShapeDtypeStruct((B_TOTAL, D_MODEL), jnp.float32),
    mesh=plsc.VectorSubcoreMesh(core_axis_name="c", subcore_axis_name="s"),
    scratch_types=[
        pltpu.VMEM((N_CHUNKS, CHUNK), jnp.int32),
        [pltpu.VMEM((CHUNK, D_MODEL), jnp.float32) for _ in range(NBUF)],
        [pltpu.SemaphoreType.DMA for _ in range(NBUF)],
        [pltpu.SemaphoreType.DMA for _ in range(NBUF)],
    ],
    compiler_params=pltpu.CompilerParams(use_tc_tiling_on_sc=False),
)


def kernel(x, table):
    x2d = x.reshape(B_TOTAL // CHUNK, CHUNK).astype(jnp.int32)
    out = _emb(x2d, table)
    return out.reshape(N_ROWS, N_COLS, D_MODEL)


# speedup vs baseline: 3.1318x; 1.2902x over previous
import jax
import jax.numpy as jnp
from jax import lax
from jax.experimental import pallas as pl
from jax.experimental.pallas import tpu as pltpu
from jax.experimental.pallas import tpu_sc as plsc

D_MODEL = 64
N_ROWS = 4096
N_COLS = 50
B_TOTAL = N_ROWS * N_COLS
NUM_WORKERS = 32
PER_WORKER = B_TOTAL // NUM_WORKERS
CHUNK = 128
N_CHUNKS = PER_WORKER // CHUNK
LANES = 16
NBUF = 10
LAG = 3
GROUPS = N_CHUNKS // NBUF
ROW_UNROLL = 2


def _tanh_lanes(z):
    e = jnp.exp(jnp.abs(z) * -2.0)
    r = (1.0 - e) / (1.0 + e)
    return jnp.sign(z) * r


def _tanh_buf(buf):
    def row_body(i, carry):
        for u in range(ROW_UNROLL):
            r = i * ROW_UNROLL + u
            for j in range(D_MODEL // LANES):
                z = buf[r, pl.ds(j * LANES, LANES)]
                buf[r, pl.ds(j * LANES, LANES)] = _tanh_lanes(z)
        return carry

    lax.fori_loop(0, CHUNK // ROW_UNROLL, row_body, 0)


def _emb_body(x_hbm, table_hbm, out_hbm, idx_all, bufs, gsems, wsems):
    wid = lax.axis_index("s") * 2 + lax.axis_index("c")
    base = wid * PER_WORKER

    pltpu.sync_copy(x_hbm.at[pl.ds(wid * N_CHUNKS, N_CHUNKS)], idx_all)

    for b in range(NBUF):
        pltpu.async_copy(table_hbm.at[idx_all.at[b]], bufs[b], gsems[b])

    def group_body(g, carry):
        for b in range(NBUF):
            c = g * NBUF + b
            pltpu.make_async_copy(
                table_hbm.at[idx_all.at[c]], bufs[b], gsems[b]).wait()
            _tanh_buf(bufs[b])
            pltpu.async_copy(
                bufs[b], out_hbm.at[pl.ds(base + c * CHUNK, CHUNK)], wsems[b])

            b2 = (b - LAG) % NBUF
            cw = c - LAG
            cg = c - LAG + NBUF

            @pl.when(jnp.logical_and(cw >= 0, cg <= N_CHUNKS - 1))
            def _():
                pltpu.make_async_copy(
                    bufs[b2], out_hbm.at[pl.ds(base + cw * CHUNK, CHUNK)],
                    wsems[b2]).wait()
                pltpu.async_copy(
                    table_hbm.at[idx_all.at[cg]], bufs[b2], gsems[b2])

        return carry

    lax.fori_loop(0, GROUPS, group_body, 0)

    for b in range(NBUF):
        c = N_CHUNKS - NBUF + b
        b2 = c % NBUF
        pltpu.make_async_copy(
            bufs[b2], out_hbm.at[pl.ds(base + c * CHUNK, CHUNK)],
            wsems[b2]).wait()


_emb = pl.kernel(
    _emb_body,
    out_type=jax.ShapeDtypeStruct((B_TOTAL, D_MODEL), jnp.float32),
    mesh=plsc.VectorSubcoreMesh(core_axis_name="c", subcore_axis_name="s"),
    scratch_types=[
        pltpu.VMEM((N_CHUNKS, CHUNK), jnp.int32),
        [pltpu.VMEM((CHUNK, D_MODEL), jnp.float32) for _ in range(NBUF)],
        [pltpu.SemaphoreType.DMA for _ in range(NBUF)],
        [pltpu.SemaphoreType.DMA for _ in range(NBUF)],
    ],
    compiler_params=pltpu.CompilerParams(use_tc_tiling_on_sc=False),
)


def kernel(x, table):
    x2d = x.reshape(B_TOTAL // CHUNK, CHUNK).astype(jnp.int32)
    out = _emb(x2d, table)
    return out.reshape(N_ROWS, N_COLS, D_MODEL)


# speedup vs baseline: 3.5235x; 1.1251x over previous
import jax
import jax.numpy as jnp
from jax import lax
from jax.experimental import pallas as pl
from jax.experimental.pallas import tpu as pltpu
from jax.experimental.pallas import tpu_sc as plsc

D_MODEL = 64
N_ROWS = 4096
N_COLS = 50
B_TOTAL = N_ROWS * N_COLS
NUM_WORKERS = 32
PER_WORKER = B_TOTAL // NUM_WORKERS
CHUNK = 128
N_CHUNKS = PER_WORKER // CHUNK
LANES = 16
NBUF = 10
LAG = 3
GROUPS = N_CHUNKS // NBUF
ROW_UNROLL = 4


def _tanh_lanes(z):
    e = jnp.exp(z * 2.0)
    return 1.0 - 2.0 / (1.0 + e)


def _tanh_buf(buf):
    def row_body(i, carry):
        for u in range(ROW_UNROLL):
            r = i * ROW_UNROLL + u
            for j in range(D_MODEL // LANES):
                z = buf[r, pl.ds(j * LANES, LANES)]
                buf[r, pl.ds(j * LANES, LANES)] = _tanh_lanes(z)
        return carry

    lax.fori_loop(0, CHUNK // ROW_UNROLL, row_body, 0)


def _emb_body(x_hbm, table_hbm, out_hbm, idx_all, bufs, gsems, wsems):
    wid = lax.axis_index("s") * 2 + lax.axis_index("c")
    base = wid * PER_WORKER

    pltpu.sync_copy(x_hbm.at[pl.ds(wid * N_CHUNKS, N_CHUNKS)], idx_all)

    for b in range(NBUF):
        pltpu.async_copy(table_hbm.at[idx_all.at[b]], bufs[b], gsems[b])

    def group_body(g, carry):
        for b in range(NBUF):
            c = g * NBUF + b
            pltpu.make_async_copy(
                table_hbm.at[idx_all.at[c]], bufs[b], gsems[b]).wait()
            _tanh_buf(bufs[b])
            pltpu.async_copy(
                bufs[b], out_hbm.at[pl.ds(base + c * CHUNK, CHUNK)], wsems[b])

            b2 = (b - LAG) % NBUF
            cw = c - LAG
            cg = c - LAG + NBUF

            @pl.when(jnp.logical_and(cw >= 0, cg <= N_CHUNKS - 1))
            def _():
                pltpu.make_async_copy(
                    bufs[b2], out_hbm.at[pl.ds(base + cw * CHUNK, CHUNK)],
                    wsems[b2]).wait()
                pltpu.async_copy(
                    table_hbm.at[idx_all.at[cg]], bufs[b2], gsems[b2])

        return carry

    lax.fori_loop(0, GROUPS, group_body, 0)

    for b in range(NBUF):
        c = N_CHUNKS - NBUF + b
        b2 = c % NBUF
        pltpu.make_async_copy(
            bufs[b2], out_hbm.at[pl.ds(base + c * CHUNK, CHUNK)],
            wsems[b2]).wait()


_emb = pl.kernel(
    _emb_body,
    out_type=jax.ShapeDtypeStruct((B_TOTAL, D_MODEL), jnp.float32),
    mesh=plsc.VectorSubcoreMesh(core_axis_name="c", subcore_axis_name="s"),
    scratch_types=[
        pltpu.VMEM((N_CHUNKS, CHUNK), jnp.int32),
        [pltpu.VMEM((CHUNK, D_MODEL), jnp.float32) for _ in range(NBUF)],
        [pltpu.SemaphoreType.DMA for _ in range(NBUF)],
        [pltpu.SemaphoreType.DMA for _ in range(NBUF)],
    ],
    compiler_params=pltpu.CompilerParams(use_tc_tiling_on_sc=False),
)


def kernel(x, table):
    x2d = x.reshape(B_TOTAL // CHUNK, CHUNK).astype(jnp.int32)
    out = _emb(x2d, table)
    return out.reshape(N_ROWS, N_COLS, D_MODEL)
